# baseline (speedup 1.0000x reference)
import functools

import numpy as np
import jax
import jax.numpy as jnp
from jax import lax
from jax.experimental import pallas as pl
from jax.experimental.pallas import tpu as pltpu
from jax.experimental.pallas import tpu_sc as plsc

_FIELD_DIMS = [1000] * 26
_F = 26
_FEAT = sum(_FIELD_DIMS)
_D = 16
_PAIRS = _F * (_F - 1) // 2
_INTER = _PAIRS * _D
_HROWS = 41
_B = 4096
_OFFS = np.asarray([0, *np.cumsum(_FIELD_DIMS)[:-1]], dtype=np.int32)

_NIDX = _F * _F
_NIDX_PAD = 688
_CHUNKS = (128, 128, 128, 128, 128, 48)

_NW = 32
_BPW = _B // _NW


def _pair_perm():
    def old_idx(f, g):
        f, g = min(f, g), max(f, g)
        return f * (2 * _F - f - 1) // 2 + (g - f - 1)
    perm = []
    for r in range(_F - 1):
        for m in range(13):
            if m == 0:
                a, b = r, _F - 1
            else:
                a = (r + m) % (_F - 1)
                b = (r - m) % (_F - 1)
            perm.append(old_idx(a, b))
    assert sorted(perm) == list(range(_PAIRS))
    return np.asarray(perm, dtype=np.int32)


_PERM = _pair_perm()


def _sc_make():
    mesh = plsc.VectorSubcoreMesh(core_axis_name="c", subcore_axis_name="s")

    @functools.partial(
        pl.kernel,
        mesh=mesh,
        compiler_params=pltpu.CompilerParams(
            use_tc_tiling_on_sc=False, needs_layout_passes=False),
        out_type=[
            jax.ShapeDtypeStruct((_B * _HROWS, 128), jnp.float32),
        ],
        scratch_types=[
            pltpu.VMEM((4, _NIDX_PAD), jnp.int32),
            pltpu.VMEM((_F, _BPW), jnp.int32),
            pltpu.VMEM((4, _NIDX_PAD, _D), jnp.float32),
            pltpu.VMEM((2, _HROWS, 128), jnp.float32),
            pltpu.VMEM((4, 32, _D), jnp.float32),
            pltpu.SemaphoreType.DMA,
            pltpu.SemaphoreType.DMA,
            pltpu.SemaphoreType.DMA,
            pltpu.SemaphoreType.DMA,
            pltpu.SemaphoreType.DMA,
            pltpu.SemaphoreType.DMA,
        ],
    )
    def sc_interactions(emb_hbm, xoffT_hbm, wpad_hbm, h_hbm,
                        ix_v, idx_v, R_v, h_v, w_v,
                        gsem0, gsem1, gsem2, gsem3, wsem0, wsem1):
        wid = lax.axis_index("s") * 2 + lax.axis_index("c")
        base = wid * _BPW
        gsems = (gsem0, gsem1, gsem2, gsem3)
        wsems = (wsem0, wsem1)

        pltpu.sync_copy(xoffT_hbm.at[:, pl.ds(base, _BPW)], idx_v)

        zero16 = jnp.zeros((16,), jnp.float32)
        for slot in (0, 1):
            for j in range(3):
                h_v[slot, _HROWS - 1, pl.ds(80 + j * 16, 16)] = zero16

        lane_iota = lax.iota(jnp.int32, 16)

        def build_ix(b, slot):
            bvec = jnp.zeros((16,), jnp.int32) + b
            def chunk(k, _):
                vj = lane_iota + k * 16
                vf = jnp.minimum(lax.div(vj, jnp.int32(_F)), _F - 1)
                vg = jnp.minimum(vj - vf * _F, _F - 1)
                xv = plsc.load_gather(idx_v, [vg, bvec])
                ix_v[slot, pl.ds(pl.multiple_of(k * 16, 16), 16)] = (
                    xv + vf * _FEAT)
                return 0
            lax.fori_loop(0, _NIDX_PAD // 16, chunk, 0)

        def fire_gather(slot, gsem):
            off = 0
            for sz in _CHUNKS:
                pltpu.async_copy(
                    emb_hbm.at[ix_v.at[slot, pl.ds(off, sz)]],
                    R_v.at[slot, pl.ds(off, sz)], gsem)
                off += sz
            pltpu.async_copy(
                wpad_hbm.at[ix_v.at[slot, pl.ds(0, 32)]],
                w_v.at[slot], gsem)

        def wait_gather(slot, gsem):
            off = 0
            for sz in _CHUNKS:
                pltpu.make_async_copy(
                    emb_hbm.at[ix_v.at[slot, pl.ds(off, sz)]],
                    R_v.at[slot, pl.ds(off, sz)], gsem).wait()
                off += sz
            pltpu.make_async_copy(
                wpad_hbm.at[ix_v.at[slot, pl.ds(0, 32)]],
                w_v.at[slot], gsem).wait()

        for s in range(4):
            build_ix(s, s)
        for s in range(3):
            fire_gather(s, gsems[s])

        def half_step(b, slot, hs, wsem):
            wait_gather(slot, gsems[slot])
            @pl.when(b + 3 < _BPW)
            def _():
                fire_gather((slot + 3) % 4, gsems[(slot + 3) % 4])
            @pl.when(b + 4 < _BPW)
            def _():
                build_ix(b + 4, slot)
            @pl.when(b >= 2)
            def _():
                pltpu.make_async_copy(
                    h_v.at[hs],
                    h_hbm.at[pl.ds((base + b - 2) * _HROWS, _HROWS)],
                    wsem).wait()

            def round_loop(r, _):
                p0 = r * (13 * 16)
                for m in range(13):
                    if m == 0:
                        a, bb = r, _F - 1
                    else:
                        a = r + m
                        a = jnp.where(a >= _F - 1, a - (_F - 1), a)
                        bb = r - m + (_F - 1)
                        bb = jnp.where(bb >= _F - 1, bb - (_F - 1), bb)
                    va = R_v[slot, a * _F + bb, :]
                    vb = R_v[slot, bb * _F + a, :]
                    p = p0 + m * 16
                    pr = lax.shift_right_logical(p, 7)
                    pc = lax.bitwise_and(p, 127)
                    h_v[hs, pr, pl.ds(pl.multiple_of(pc, 16), 16)] = va * vb
                return 0
            lax.fori_loop(0, _F - 1, round_loop, 0)

            def w_loop(f, acc):
                return acc + w_v[slot, f, :]
            h_v[hs, _HROWS - 1, pl.ds(80, 16)] = lax.fori_loop(
                0, _F, w_loop, jnp.zeros((16,), jnp.float32))

            pltpu.async_copy(
                h_v.at[hs],
                h_hbm.at[pl.ds((base + b) * _HROWS, _HROWS)], wsem)

        def iter_body(i, _):
            b0 = 4 * i
            for k in range(4):
                half_step(b0 + k, k, k % 2, wsems[k % 2])
            return 0
        lax.fori_loop(0, _BPW // 4, iter_body, 0)

        pltpu.make_async_copy(
            h_v.at[0],
            h_hbm.at[pl.ds((base + _BPW - 2) * _HROWS, _HROWS)], wsem0).wait()
        pltpu.make_async_copy(
            h_v.at[1],
            h_hbm.at[pl.ds((base + _BPW - 1) * _HROWS, _HROWS)], wsem1).wait()

    return sc_interactions


_sc_interactions = _sc_make()

_BT = 512


def _mlp_body(h_ref, W1_ref, b1_ref, W2_ref, b2_ref, W3_ref, b3_ref, e_ref,
              out_ref):
    h2 = h_ref[...].reshape(_BT, _HROWS * 128)
    a1 = jnp.dot(h2, W1_ref[...], preferred_element_type=jnp.float32)
    a1 = jnp.maximum(a1 + b1_ref[...], 0.0)
    a2 = jnp.dot(a1, W2_ref[...], preferred_element_type=jnp.float32)
    a2 = jnp.maximum(a2 + b2_ref[...], 0.0)
    a3 = jnp.dot(a2, W3_ref[...], preferred_element_type=jnp.float32)
    fo = jnp.dot(h2, e_ref[...], preferred_element_type=jnp.float32)
    out_ref[...] = a3 + fo + b3_ref[...]


_mlp_call = pl.pallas_call(
    _mlp_body,
    grid=(_B // _BT,),
    in_specs=[
        pl.BlockSpec((_BT * _HROWS, 128), lambda i: (i, 0)),
        pl.BlockSpec((_HROWS * 128, 64), lambda i: (0, 0)),
        pl.BlockSpec((1, 64), lambda i: (0, 0)),
        pl.BlockSpec((64, 32), lambda i: (0, 0)),
        pl.BlockSpec((1, 32), lambda i: (0, 0)),
        pl.BlockSpec((32, 1), lambda i: (0, 0)),
        pl.BlockSpec((1, 1), lambda i: (0, 0)),
        pl.BlockSpec((_HROWS * 128, 1), lambda i: (0, 0)),
    ],
    out_specs=pl.BlockSpec((_BT, 1), lambda i: (i, 0)),
    out_shape=jax.ShapeDtypeStruct((_B, 1), jnp.float32),
)


def kernel(x, emb, w_lin, b_lin, W1, b1, W2, b2, W3, b3):
    x_off = x + jnp.asarray(_OFFS)[None, :]
    emb_flat = emb.reshape(_F * _FEAT, _D)
    w_pad = jnp.concatenate(
        [w_lin.reshape(_FEAT, 1), jnp.zeros((_FEAT, 15), jnp.float32)], axis=1)
    (h,) = _sc_interactions(emb_flat, x_off.T, w_pad)
    W1perm = W1.reshape(_PAIRS, _D, 64)[_PERM].reshape(_INTER, 64)
    W1p = jnp.concatenate(
        [W1perm, jnp.zeros((_HROWS * 128 - _INTER, 64), jnp.float32)], axis=0)
    e = jnp.zeros((_HROWS * 128, 1), jnp.float32).at[_INTER, 0].set(1.0)
    out = _mlp_call(h, W1p, b1.reshape(1, 64), W2, b2.reshape(1, 32),
                    W3, (b3 + b_lin).reshape(1, 1), e)
    return out[:, 0]
